# baseline (speedup 1.0000x reference)
import functools

import jax
import jax.numpy as jnp
from jax import lax
from jax.experimental import pallas as pl
from jax.experimental.pallas import tpu as pltpu
from jax.experimental.pallas import tpu_sc as plsc

_N = 10000
_E = 320000
_NC = 2
_NS = 16
_NW = _NC * _NS
_EPT = _E // _NW
_K = 80
_NCH = _EPT // _K
_NPAN = 5
_CPP = _NCH // _NPAN
_NP = 10240
_RPT = _NP // _NS
_R = 1000

_mesh = plsc.VectorSubcoreMesh(core_axis_name="c", subcore_axis_name="s")


@functools.partial(
    pl.kernel,
    mesh=_mesh,
    out_type=jax.ShapeDtypeStruct((_NC, _NP, 16), jnp.float32),
    scratch_types=[
        pltpu.VMEM((_NCH, _K), jnp.int32),
        pltpu.VMEM((_K, 16), jnp.float32),
        pltpu.VMEM_SHARED((_NP, 16), jnp.float32),
    ],
)
def _deg_kernel(dst_hbm, ones_hbm, zeros_hbm, out_hbm, dstv, onesv, shared):
    c = lax.axis_index("c")
    s = lax.axis_index("s")
    w = c * _NS + s
    pltpu.sync_copy(dst_hbm.at[w], dstv)
    pltpu.sync_copy(ones_hbm, onesv)
    pltpu.sync_copy(zeros_hbm, shared.at[pl.ds(s * _RPT, _RPT)])
    plsc.subcore_barrier()

    @pl.loop(0, _NCH)
    def _acc(j):
        pltpu.sync_copy(onesv, shared.at[dstv.at[j]], add=True)

    plsc.subcore_barrier()
    pltpu.sync_copy(shared.at[pl.ds(s * _RPT, _RPT)],
                    out_hbm.at[c, pl.ds(s * _RPT, _RPT)])


def _make_agg(d):
    @functools.partial(
        pl.kernel,
        mesh=_mesh,
        out_type=jax.ShapeDtypeStruct((_NC, _NP, d), jnp.float32),
        scratch_types=[
            pltpu.VMEM((_CPP, _K), jnp.int32),
            pltpu.VMEM((_CPP, _K), jnp.int32),
            pltpu.VMEM((4, _K, d), jnp.float32),
            pltpu.VMEM_SHARED((_NP, d), jnp.float32),
            pltpu.SemaphoreType.DMA,
            pltpu.SemaphoreType.DMA,
            pltpu.SemaphoreType.DMA,
            pltpu.SemaphoreType.DMA,
            pltpu.SemaphoreType.DMA,
            pltpu.SemaphoreType.DMA,
            pltpu.SemaphoreType.DMA,
            pltpu.SemaphoreType.DMA,
        ],
    )
    def agg(y_hbm, src_hbm, dst_hbm, zeros_hbm, out_hbm,
            srcp, dstp, bufs, shared,
            g0, g1, g2, g3, s0, s1, s2, s3):
        c = lax.axis_index("c")
        s = lax.axis_index("s")
        w = c * _NS + s
        gsem = (g0, g1, g2, g3)
        ssem = (s0, s1, s2, s3)
        pltpu.sync_copy(zeros_hbm, shared.at[pl.ds(s * _RPT, _RPT)])
        plsc.subcore_barrier()

        @pl.loop(0, _NPAN)
        def _panel(p):
            pltpu.sync_copy(src_hbm.at[w, p], srcp)
            pltpu.sync_copy(dst_hbm.at[w, p], dstp)
            for b in range(4):
                pltpu.async_copy(y_hbm.at[srcp.at[b]], bufs.at[b], gsem[b])
            for b in range(4):
                pltpu.make_async_copy(
                    y_hbm.at[srcp.at[b]], bufs.at[b], gsem[b]).wait()
                pltpu.async_copy(bufs.at[b], shared.at[dstp.at[b]],
                                 ssem[b], add=True)

            @pl.loop(4, _CPP - 1, step=4)
            def _pipe(i):
                for b in range(4):
                    pltpu.make_async_copy(
                        bufs.at[b], shared.at[dstp.at[i - 4 + b]],
                        ssem[b]).wait()
                    pltpu.async_copy(y_hbm.at[srcp.at[i + b]], bufs.at[b],
                                     gsem[b])
                for b in range(4):
                    pltpu.make_async_copy(
                        y_hbm.at[srcp.at[i + b]], bufs.at[b], gsem[b]).wait()
                    pltpu.async_copy(bufs.at[b], shared.at[dstp.at[i + b]],
                                     ssem[b], add=True)

            for b in range(4):
                pltpu.make_async_copy(
                    bufs.at[b], shared.at[dstp.at[_CPP - 5 + b]],
                    ssem[b]).wait()
            pltpu.async_copy(y_hbm.at[srcp.at[_CPP - 1]], bufs.at[0], gsem[0])
            pltpu.make_async_copy(
                y_hbm.at[srcp.at[_CPP - 1]], bufs.at[0], gsem[0]).wait()
            pltpu.sync_copy(bufs.at[0], shared.at[dstp.at[_CPP - 1]], add=True)

        plsc.subcore_barrier()
        pltpu.sync_copy(shared.at[pl.ds(s * _RPT, _RPT)],
                        out_hbm.at[c, pl.ds(s * _RPT, _RPT)])

    return agg


_agg128 = _make_agg(128)


def _norm_from(degp):
    deg = degp[0, :, 0:1] + degp[1, :, 0:1]
    return jnp.where(deg > 0, lax.rsqrt(jnp.maximum(deg, 1.0)), 0.0)


def _tc1_body(x_ref, w_ref, degp_ref, y_ref):
    norm = _norm_from(degp_ref[...])
    y_ref[...] = norm * jnp.dot(x_ref[...], w_ref[...],
                                preferred_element_type=jnp.float32)


def _tc_mid_body(p_ref, degp_ref, b_ref, w_ref, y_ref):
    a = p_ref[...]
    norm = _norm_from(degp_ref[...])
    h = (a[0] + a[1]) * norm + b_ref[...]
    y_ref[...] = norm * jnp.dot(h, w_ref[...],
                                preferred_element_type=jnp.float32)


def _tc_scale_body(p_ref, degp_ref, b_ref, y_ref):
    a = p_ref[...]
    norm = _norm_from(degp_ref[...])
    y_ref[...] = norm * ((a[0] + a[1]) * norm + b_ref[...])


def _tc_final_body(p_ref, degp_ref, b_ref, w_ref, out_ref):
    a = p_ref[...]
    norm = _norm_from(degp_ref[...])
    o = jnp.dot((a[0] + a[1]) * norm, w_ref[...],
                preferred_element_type=jnp.float32) + b_ref[...]
    m = jnp.max(o, axis=1, keepdims=True)
    ls = jnp.log(jnp.sum(jnp.exp(o - m), axis=1, keepdims=True))
    out_ref[...] = o - m - ls


_x_spec = pl.BlockSpec((_R, 128), lambda i: (i, 0))
_p_spec = pl.BlockSpec((_NC, _R, 128), lambda i: (0, i, 0))
_degp_spec = pl.BlockSpec((_NC, _R, 16), lambda i: (0, i, 0))


def _tc1(x, w1, degp):
    return pl.pallas_call(
        _tc1_body,
        grid=(_N // _R,),
        in_specs=[_x_spec, pl.BlockSpec((128, 128), lambda i: (0, 0)),
                  _degp_spec],
        out_specs=_x_spec,
        out_shape=jax.ShapeDtypeStruct((_N, 128), jnp.float32),
    )(x, w1, degp)


def _tc_mid(p, degp, b, w):
    return pl.pallas_call(
        _tc_mid_body,
        grid=(_N // _R,),
        in_specs=[_p_spec, _degp_spec,
                  pl.BlockSpec((1, 128), lambda i: (0, 0)),
                  pl.BlockSpec((128, 128), lambda i: (0, 0))],
        out_specs=_x_spec,
        out_shape=jax.ShapeDtypeStruct((_N, 128), jnp.float32),
    )(p, degp, b, w)


def _tc_scale(p, degp, b):
    return pl.pallas_call(
        _tc_scale_body,
        grid=(_N // _R,),
        in_specs=[_p_spec, _degp_spec,
                  pl.BlockSpec((1, 128), lambda i: (0, 0))],
        out_specs=_x_spec,
        out_shape=jax.ShapeDtypeStruct((_N, 128), jnp.float32),
    )(p, degp, b)


def _tc_final(p, degp, b, w):
    return pl.pallas_call(
        _tc_final_body,
        grid=(_N // _R,),
        in_specs=[_p_spec, _degp_spec,
                  pl.BlockSpec((1, 64), lambda i: (0, 0)),
                  pl.BlockSpec((128, 64), lambda i: (0, 0))],
        out_specs=pl.BlockSpec((_R, 64), lambda i: (i, 0)),
        out_shape=jax.ShapeDtypeStruct((_N, 64), jnp.float32),
    )(p, degp, b, w)


def kernel(features, edge_index, W1, b1, W2, b2, W3, b3):
    src = edge_index[0].reshape(_NW, _NPAN, _CPP, _K)
    dst = edge_index[1].reshape(_NW, _NPAN, _CPP, _K)
    dst2 = edge_index[1].reshape(_NW, _NCH, _K)
    ones16 = jnp.ones((_K, 16), jnp.float32)
    z16 = jnp.zeros((_RPT, 16), jnp.float32)
    z128 = jnp.zeros((_RPT, 128), jnp.float32)

    degp = _deg_kernel(dst2, ones16, z16)
    y1 = _tc1(features, W1, degp)
    p1 = _agg128(y1, src, dst, z128)
    y2 = _tc_mid(p1, degp, b1.reshape(1, 128), W2)
    p2 = _agg128(y2, src, dst, z128)
    y3 = _tc_scale(p2, degp, b2.reshape(1, 128))
    p3 = _agg128(y3, src, dst, z128)
    return _tc_final(p3, degp, b3.reshape(1, 64), W3)


# speedup vs baseline: 1.0142x; 1.0142x over previous
import functools

import jax
import jax.numpy as jnp
from jax import lax
from jax.experimental import pallas as pl
from jax.experimental.pallas import tpu as pltpu
from jax.experimental.pallas import tpu_sc as plsc

_N = 10000
_E = 320000
_NC = 2
_NS = 16
_NW = _NC * _NS
_EPT = _E // _NW
_K = 80
_NCH = _EPT // _K
_NPAN = 5
_CPP = _NCH // _NPAN
_NP = 10240
_RPT = _NP // _NS
_R = 1000

_mesh = plsc.VectorSubcoreMesh(core_axis_name="c", subcore_axis_name="s")


@functools.partial(
    pl.kernel,
    mesh=_mesh,
    out_type=jax.ShapeDtypeStruct((_NC, _NP, 16), jnp.float32),
    scratch_types=[
        pltpu.VMEM((_NCH, _K), jnp.int32),
        pltpu.VMEM((_K, 16), jnp.float32),
        pltpu.VMEM_SHARED((_NP, 16), jnp.float32),
        pltpu.SemaphoreType.DMA,
    ],
)
def _deg_kernel(dst_hbm, ones_hbm, zeros_hbm, out_hbm, dstv, onesv, shared,
                sem):
    c = lax.axis_index("c")
    s = lax.axis_index("s")
    w = c * _NS + s
    pltpu.sync_copy(dst_hbm.at[w], dstv)
    pltpu.sync_copy(ones_hbm, onesv)
    pltpu.sync_copy(zeros_hbm, shared.at[pl.ds(s * _RPT, _RPT)])
    plsc.subcore_barrier()

    @pl.loop(0, _NCH)
    def _acc(j):
        pltpu.async_copy(onesv, shared.at[dstv.at[j]], sem, add=True)

    @pl.loop(0, _NCH)
    def _drain(j):
        pltpu.make_async_copy(onesv, shared.at[dstv.at[j]], sem).wait()

    plsc.subcore_barrier()
    pltpu.sync_copy(shared.at[pl.ds(s * _RPT, _RPT)],
                    out_hbm.at[c, pl.ds(s * _RPT, _RPT)])


def _make_agg(d):
    @functools.partial(
        pl.kernel,
        mesh=_mesh,
        out_type=jax.ShapeDtypeStruct((_NC, _NP, d), jnp.float32),
        scratch_types=[
            pltpu.VMEM((_CPP, _K), jnp.int32),
            pltpu.VMEM((_CPP, _K), jnp.int32),
            pltpu.VMEM((4, _K, d), jnp.float32),
            pltpu.VMEM_SHARED((_NP, d), jnp.float32),
            pltpu.SemaphoreType.DMA,
            pltpu.SemaphoreType.DMA,
            pltpu.SemaphoreType.DMA,
            pltpu.SemaphoreType.DMA,
            pltpu.SemaphoreType.DMA,
            pltpu.SemaphoreType.DMA,
            pltpu.SemaphoreType.DMA,
            pltpu.SemaphoreType.DMA,
        ],
    )
    def agg(y_hbm, src_hbm, dst_hbm, zeros_hbm, out_hbm,
            srcp, dstp, bufs, shared,
            g0, g1, g2, g3, s0, s1, s2, s3):
        c = lax.axis_index("c")
        s = lax.axis_index("s")
        w = c * _NS + s
        gsem = (g0, g1, g2, g3)
        ssem = (s0, s1, s2, s3)
        pltpu.sync_copy(zeros_hbm, shared.at[pl.ds(s * _RPT, _RPT)])
        plsc.subcore_barrier()

        @pl.loop(0, _NPAN)
        def _panel(p):
            pltpu.sync_copy(src_hbm.at[w, p], srcp)
            pltpu.sync_copy(dst_hbm.at[w, p], dstp)
            for b in range(4):
                pltpu.async_copy(y_hbm.at[srcp.at[b]], bufs.at[b], gsem[b])
            for b in range(4):
                pltpu.make_async_copy(
                    y_hbm.at[srcp.at[b]], bufs.at[b], gsem[b]).wait()
                pltpu.async_copy(bufs.at[b], shared.at[dstp.at[b]],
                                 ssem[b], add=True)

            @pl.loop(4, _CPP - 1, step=4)
            def _pipe(i):
                for b in range(4):
                    pltpu.make_async_copy(
                        bufs.at[b], shared.at[dstp.at[i - 4 + b]],
                        ssem[b]).wait()
                    pltpu.async_copy(y_hbm.at[srcp.at[i + b]], bufs.at[b],
                                     gsem[b])
                for b in range(4):
                    pltpu.make_async_copy(
                        y_hbm.at[srcp.at[i + b]], bufs.at[b], gsem[b]).wait()
                    pltpu.async_copy(bufs.at[b], shared.at[dstp.at[i + b]],
                                     ssem[b], add=True)

            for b in range(4):
                pltpu.make_async_copy(
                    bufs.at[b], shared.at[dstp.at[_CPP - 5 + b]],
                    ssem[b]).wait()
            pltpu.async_copy(y_hbm.at[srcp.at[_CPP - 1]], bufs.at[0], gsem[0])
            pltpu.make_async_copy(
                y_hbm.at[srcp.at[_CPP - 1]], bufs.at[0], gsem[0]).wait()
            pltpu.sync_copy(bufs.at[0], shared.at[dstp.at[_CPP - 1]], add=True)

        plsc.subcore_barrier()
        pltpu.sync_copy(shared.at[pl.ds(s * _RPT, _RPT)],
                        out_hbm.at[c, pl.ds(s * _RPT, _RPT)])

    return agg


_agg128 = _make_agg(128)


def _norm_from(degp):
    deg = degp[0, :, 0:1] + degp[1, :, 0:1]
    return jnp.where(deg > 0, lax.rsqrt(jnp.maximum(deg, 1.0)), 0.0)


def _tc1_body(x_ref, w_ref, degp_ref, y_ref):
    norm = _norm_from(degp_ref[...])
    y_ref[...] = norm * jnp.dot(x_ref[...], w_ref[...],
                                preferred_element_type=jnp.float32)


def _tc_mid_body(p_ref, degp_ref, b_ref, w_ref, y_ref):
    a = p_ref[...]
    norm = _norm_from(degp_ref[...])
    h = (a[0] + a[1]) * norm + b_ref[...]
    y_ref[...] = norm * jnp.dot(h, w_ref[...],
                                preferred_element_type=jnp.float32)


def _tc_scale_body(p_ref, degp_ref, b_ref, y_ref):
    a = p_ref[...]
    norm = _norm_from(degp_ref[...])
    y_ref[...] = norm * ((a[0] + a[1]) * norm + b_ref[...])


def _tc_final_body(p_ref, degp_ref, b_ref, w_ref, out_ref):
    a = p_ref[...]
    norm = _norm_from(degp_ref[...])
    o = jnp.dot((a[0] + a[1]) * norm, w_ref[...],
                preferred_element_type=jnp.float32) + b_ref[...]
    m = jnp.max(o, axis=1, keepdims=True)
    ls = jnp.log(jnp.sum(jnp.exp(o - m), axis=1, keepdims=True))
    out_ref[...] = o - m - ls


_x_spec = pl.BlockSpec((_R, 128), lambda i: (i, 0))
_p_spec = pl.BlockSpec((_NC, _R, 128), lambda i: (0, i, 0))
_degp_spec = pl.BlockSpec((_NC, _R, 16), lambda i: (0, i, 0))


def _tc1(x, w1, degp):
    return pl.pallas_call(
        _tc1_body,
        grid=(_N // _R,),
        in_specs=[_x_spec, pl.BlockSpec((128, 128), lambda i: (0, 0)),
                  _degp_spec],
        out_specs=_x_spec,
        out_shape=jax.ShapeDtypeStruct((_N, 128), jnp.float32),
    )(x, w1, degp)


def _tc_mid(p, degp, b, w):
    return pl.pallas_call(
        _tc_mid_body,
        grid=(_N // _R,),
        in_specs=[_p_spec, _degp_spec,
                  pl.BlockSpec((1, 128), lambda i: (0, 0)),
                  pl.BlockSpec((128, 128), lambda i: (0, 0))],
        out_specs=_x_spec,
        out_shape=jax.ShapeDtypeStruct((_N, 128), jnp.float32),
    )(p, degp, b, w)


def _tc_scale(p, degp, b):
    return pl.pallas_call(
        _tc_scale_body,
        grid=(_N // _R,),
        in_specs=[_p_spec, _degp_spec,
                  pl.BlockSpec((1, 128), lambda i: (0, 0))],
        out_specs=_x_spec,
        out_shape=jax.ShapeDtypeStruct((_N, 128), jnp.float32),
    )(p, degp, b)


def _tc_final(p, degp, b, w):
    return pl.pallas_call(
        _tc_final_body,
        grid=(_N // _R,),
        in_specs=[_p_spec, _degp_spec,
                  pl.BlockSpec((1, 64), lambda i: (0, 0)),
                  pl.BlockSpec((128, 64), lambda i: (0, 0))],
        out_specs=pl.BlockSpec((_R, 64), lambda i: (i, 0)),
        out_shape=jax.ShapeDtypeStruct((_N, 64), jnp.float32),
    )(p, degp, b, w)


def kernel(features, edge_index, W1, b1, W2, b2, W3, b3):
    src = edge_index[0].reshape(_NW, _NPAN, _CPP, _K)
    dst = edge_index[1].reshape(_NW, _NPAN, _CPP, _K)
    dst2 = edge_index[1].reshape(_NW, _NCH, _K)
    ones16 = jnp.ones((_K, 16), jnp.float32)
    z16 = jnp.zeros((_RPT, 16), jnp.float32)
    z128 = jnp.zeros((_RPT, 128), jnp.float32)

    degp = _deg_kernel(dst2, ones16, z16)
    y1 = _tc1(features, W1, degp)
    p1 = _agg128(y1, src, dst, z128)
    y2 = _tc_mid(p1, degp, b1.reshape(1, 128), W2)
    p2 = _agg128(y2, src, dst, z128)
    y3 = _tc_scale(p2, degp, b2.reshape(1, 128))
    p3 = _agg128(y3, src, dst, z128)
    return _tc_final(p3, degp, b3.reshape(1, 64), W3)
